# baseline (speedup 1.0000x reference)
import jax
import jax.numpy as jnp
from jax.experimental import pallas as pl
from jax.experimental.pallas import tpu as pltpu

_ROWS = 128
_COLS = 100000
_BLOCK_ROWS = 8
_BLOCK_COLS = 12800

_ROTS = ((13, 15, 26, 6), (17, 29, 16, 24))


def _bern_kernel(p_ref, o_ref):
    ir = pl.program_id(0)
    ic = pl.program_id(1)
    R, C = p_ref.shape
    row = jax.lax.broadcasted_iota(jnp.uint32, (R, C), 0)
    col = jax.lax.broadcasted_iota(jnp.uint32, (R, C), 1)
    base = (ir * R * _COLS + ic * _BLOCK_COLS).astype(jnp.uint32)
    idx = base + row * jnp.uint32(_COLS) + col

    k0 = jnp.uint32(0)
    k1 = jnp.uint32(42)
    ks = (k0, k1, k0 ^ k1 ^ jnp.uint32(0x1BD11BDA))
    x0 = jnp.full_like(idx, k0)
    x1 = idx + k1
    for i in range(5):
        for rot in _ROTS[i % 2]:
            x0 = x0 + x1
            x1 = (x1 << rot) | (x1 >> (32 - rot))
            x1 = x1 ^ x0
        x0 = x0 + ks[(i + 1) % 3]
        x1 = x1 + ks[(i + 2) % 3] + jnp.uint32(i + 1)

    bits = x0 ^ x1
    fb = (bits >> jnp.uint32(9)) | jnp.uint32(0x3F800000)
    u = jax.lax.bitcast_convert_type(fb, jnp.float32) - jnp.float32(1.0)
    o_ref[...] = (u < p_ref[...]).astype(jnp.float32)


def kernel(probs):
    return pl.pallas_call(
        _bern_kernel,
        grid=(_ROWS // _BLOCK_ROWS, pl.cdiv(_COLS, _BLOCK_COLS)),
        in_specs=[pl.BlockSpec((_BLOCK_ROWS, _BLOCK_COLS), lambda r, c: (r, c))],
        out_specs=pl.BlockSpec((_BLOCK_ROWS, _BLOCK_COLS), lambda r, c: (r, c)),
        out_shape=jax.ShapeDtypeStruct((_ROWS, _COLS), probs.dtype),
        compiler_params=pltpu.CompilerParams(
            dimension_semantics=("parallel", "parallel")),
    )(probs)


# speedup vs baseline: 1.0341x; 1.0341x over previous
import jax
import jax.numpy as jnp
from jax.experimental import pallas as pl
from jax.experimental.pallas import tpu as pltpu

_ROWS = 128
_COLS = 100000
_BLOCK_ROWS = 8
_N_STEPS = _ROWS // _BLOCK_ROWS

_ROTS = ((13, 15, 26, 6), (17, 29, 16, 24))


def _sample_block(p_block, step):
    R, C = p_block.shape
    row = jax.lax.broadcasted_iota(jnp.uint32, (R, C), 0)
    col = jax.lax.broadcasted_iota(jnp.uint32, (R, C), 1)
    base = (step * R * _COLS).astype(jnp.uint32)
    idx = base + row * jnp.uint32(_COLS) + col

    k0 = jnp.uint32(0)
    k1 = jnp.uint32(42)
    ks = (k0, k1, k0 ^ k1 ^ jnp.uint32(0x1BD11BDA))
    x0 = jnp.full_like(idx, k0)
    x1 = idx + k1
    for i in range(5):
        for rot in _ROTS[i % 2]:
            x0 = x0 + x1
            x1 = (x1 << rot) | (x1 >> (32 - rot))
            x1 = x1 ^ x0
        x0 = x0 + ks[(i + 1) % 3]
        x1 = x1 + ks[(i + 2) % 3] + jnp.uint32(i + 1)

    bits = x0 ^ x1
    fb = (bits >> jnp.uint32(9)) | jnp.uint32(0x3F800000)
    u = jax.lax.bitcast_convert_type(fb, jnp.float32) - jnp.float32(1.0)
    return (u < p_block).astype(jnp.float32)


def _pipelined(p_hbm, o_hbm, p_vmem, o_vmem, in_sem, out_sem):
    def in_copy(step, slot):
        return pltpu.make_async_copy(
            p_hbm.at[pl.ds(step * _BLOCK_ROWS, _BLOCK_ROWS), :],
            p_vmem.at[slot], in_sem.at[slot])

    def out_copy(step, slot):
        return pltpu.make_async_copy(
            o_vmem.at[slot],
            o_hbm.at[pl.ds(step * _BLOCK_ROWS, _BLOCK_ROWS), :],
            out_sem.at[slot])

    in_copy(0, 0).start()

    def body(step, carry):
        slot = jax.lax.rem(step, 2)

        @pl.when(step + 1 < _N_STEPS)
        def _():
            in_copy(step + 1, 1 - slot).start()

        in_copy(step, slot).wait()

        @pl.when(step >= 2)
        def _():
            out_copy(step - 2, slot).wait()

        o_vmem[slot] = _sample_block(p_vmem[slot], step)
        out_copy(step, slot).start()
        return carry

    jax.lax.fori_loop(0, _N_STEPS, body, 0)
    out_copy(_N_STEPS - 2, 0).wait()
    out_copy(_N_STEPS - 1, 1).wait()


def kernel(probs):
    return pl.pallas_call(
        _pipelined,
        in_specs=[pl.BlockSpec(memory_space=pl.ANY)],
        out_specs=pl.BlockSpec(memory_space=pl.ANY),
        out_shape=jax.ShapeDtypeStruct((_ROWS, _COLS), probs.dtype),
        scratch_shapes=[
            pltpu.VMEM((2, _BLOCK_ROWS, _COLS), jnp.float32),
            pltpu.VMEM((2, _BLOCK_ROWS, _COLS), jnp.float32),
            pltpu.SemaphoreType.DMA((2,)),
            pltpu.SemaphoreType.DMA((2,)),
        ],
    )(probs)
